# Initial kernel scaffold; baseline (speedup 1.0000x reference)
import jax
import jax.numpy as jnp
from jax.experimental import pallas as pl


def kernel(images, mask_idx, order):
    raise NotImplementedError("write your pallas kernel here")


# speedup vs baseline: 69.2494x; 69.2494x over previous
import functools

import jax
import jax.numpy as jnp
from jax import lax
from jax.experimental import pallas as pl
from jax.experimental.pallas import tpu as pltpu
from jax.experimental.pallas import tpu_sc as plsc

_B = 4
_NB = 65536
_NS = 32768
_NOPS = 32768
_D = 128
_EXT = _NS + 1

_CHUNK = 8192
_NGRP = _CHUNK // 16
_MCHUNK = 8192
_GCHUNK = 128
_NTILES = 32
_ROWS_PER_TILE = _B * _NB // _NTILES

_mesh = plsc.VectorSubcoreMesh(core_axis_name="c", subcore_axis_name="s")


def _take16(x, i):
    return x.at[i].get(mode="promise_in_bounds")


@functools.partial(
    pl.kernel,
    out_type=jax.ShapeDtypeStruct((_B * _NB,), jnp.int32),
    mesh=_mesh,
    scratch_types=[
        pltpu.VMEM((_NB,), jnp.int32),
        pltpu.VMEM((_CHUNK,), jnp.int32),
        pltpu.VMEM((_CHUNK,), jnp.int32),
        pltpu.VMEM((_MCHUNK,), jnp.int32),
    ],
    compiler_params=pltpu.CompilerParams(needs_layout_passes=False),
)
def _chain_kernel(midx_hbm, ford_hbm, tord_hbm, g_hbm, src_v, f_v, t_v, m_v):
    wid = lax.axis_index("s") * 2 + lax.axis_index("c")
    lane = lax.iota(jnp.int32, 16)

    @pl.when(wid < _B)
    def _():
        b = wid
        sent = b * _EXT + _NS

        def init_body(i, _):
            src_v[pl.ds(i * 16, 16)] = jnp.full((16,), 0, jnp.int32) + sent
            return 0

        lax.fori_loop(0, _NB // 16, init_body, 0)

        for mc in range(_NS // _MCHUNK):
            pltpu.sync_copy(midx_hbm.at[pl.ds(b * _NS + mc * _MCHUNK, _MCHUNK)], m_v)
            pbase = b * _EXT + mc * _MCHUNK

            def mask_body(i, _):
                m16 = m_v[pl.ds(i * 16, 16)]
                plsc.store_scatter(src_v, [m16], pbase + i * 16 + lane)
                return 0

            lax.fori_loop(0, _MCHUNK // 16, mask_body, 0)

        for ci in range(_NOPS // _CHUNK):
            cj = _NOPS // _CHUNK - 1 - ci
            off = b * _NOPS + cj * _CHUNK
            pltpu.sync_copy(ford_hbm.at[pl.ds(off, _CHUNK)], f_v)
            pltpu.sync_copy(tord_hbm.at[pl.ds(off, _CHUNK)], t_v)

            def group_body(gi, _):
                jg = _NGRP - 1 - gi
                f16 = jnp.flip(f_v[pl.ds(jg * 16, 16)])
                t16 = jnp.flip(t_v[pl.ds(jg * 16, 16)])
                q = lane
                found = lane < 0
                for d in range(1, 16):
                    tm = _take16(t16, jnp.maximum(lane - d, 0))
                    match = (tm == f16) & (lane >= d) & (~found)
                    q = jnp.where(match, lane - d, q)
                    found = found | match
                for _r in range(4):
                    q = _take16(q, q)
                srcvals = plsc.load_gather(src_v, [f16])
                vals = _take16(srcvals, q)
                killed = lane < 0
                for d in range(1, 16):
                    tl = _take16(t16, jnp.minimum(lane + d, 15))
                    killed = killed | ((tl == t16) & (lane < 16 - d))
                plsc.store_scatter(src_v, [t16], vals, mask=~killed)
                return 0

            lax.fori_loop(0, _NGRP, group_body, 0)

        pltpu.sync_copy(src_v, g_hbm.at[pl.ds(b * _NB, _NB)])


@functools.partial(
    pl.kernel,
    out_type=jax.ShapeDtypeStruct((_B * _NB, _D), jnp.float32),
    mesh=_mesh,
    scratch_types=[
        pltpu.VMEM((_GCHUNK,), jnp.int32),
        pltpu.VMEM((_GCHUNK, _D), jnp.float32),
        pltpu.SemaphoreType.DMA,
    ],
    compiler_params=pltpu.CompilerParams(needs_layout_passes=False),
)
def _gather_kernel(imgext_hbm, g_hbm, out_hbm, idx_v, rows_v, gsem):
    wid = lax.axis_index("s") * 2 + lax.axis_index("c")
    base = wid * _ROWS_PER_TILE
    nchunks = _ROWS_PER_TILE // _GCHUNK

    def chunk_body(c, _):
        off = base + c * _GCHUNK
        pltpu.sync_copy(g_hbm.at[pl.ds(off, _GCHUNK)], idx_v)
        pltpu.async_copy(imgext_hbm.at[idx_v], rows_v, gsem).wait()
        pltpu.sync_copy(rows_v, out_hbm.at[pl.ds(off, _GCHUNK)])
        return 0

    lax.fori_loop(0, nchunks, chunk_body, 0)


def kernel(images, mask_idx, order):
    imgext = jnp.concatenate(
        [images, jnp.zeros((_B, 1, _D), images.dtype)], axis=1
    ).reshape(_B * _EXT, _D)
    midx = mask_idx.astype(jnp.int32).reshape(-1)
    ford = order[:, 0, :].astype(jnp.int32).reshape(-1)
    tord = order[:, 1, :].astype(jnp.int32).reshape(-1)
    g = _chain_kernel(midx, ford, tord)
    out = _gather_kernel(imgext, g)
    return out.reshape(_B, _NB, _D)


# speedup vs baseline: 76.0319x; 1.0979x over previous
import functools

import jax
import jax.numpy as jnp
from jax import lax
from jax.experimental import pallas as pl
from jax.experimental.pallas import tpu as pltpu
from jax.experimental.pallas import tpu_sc as plsc

_B = 4
_NB = 65536
_NS = 32768
_NOPS = 32768
_D = 128
_EXT = _NS + 1

_NTILES = 32
_HSPAN = _B * _NOPS // _NTILES
_HGRP = _HSPAN // 16

_CHUNK = 8192
_NGRP = _CHUNK // 16
_MCHUNK = 8192
_UNROLL = 8

_GCHUNK = 128
_NSLOT = 4
_ROWS_PER_TILE = _B * _NB // _NTILES

_mesh = plsc.VectorSubcoreMesh(core_axis_name="c", subcore_axis_name="s")
_params = pltpu.CompilerParams(needs_layout_passes=False)


def _take16(x, i):
    return x.at[i].get(mode="promise_in_bounds")


@functools.partial(
    pl.kernel,
    out_type=(
        jax.ShapeDtypeStruct((_B * _NOPS,), jnp.int32),
        jax.ShapeDtypeStruct((_B * _NOPS,), jnp.int32),
        jax.ShapeDtypeStruct((_B * _NOPS,), jnp.int32),
    ),
    mesh=_mesh,
    scratch_types=[
        pltpu.VMEM((_HSPAN,), jnp.int32),
        pltpu.VMEM((_HSPAN,), jnp.int32),
        pltpu.VMEM((_HSPAN,), jnp.int32),
        pltpu.VMEM((_HSPAN,), jnp.int32),
        pltpu.VMEM((_HSPAN,), jnp.int32),
    ],
    compiler_params=_params,
)
def _hazard_kernel(ford_hbm, tord_hbm, fr_hbm, tr_hbm, qp_hbm,
                   f_v, t_v, fr_v, tr_v, qp_v):
    wid = lax.axis_index("s") * 2 + lax.axis_index("c")
    lane = lax.iota(jnp.int32, 16)
    b = wid // (_NOPS // _HSPAN)
    s0loc = (wid % (_NOPS // _HSPAN)) * _HSPAN
    joff = b * _NOPS + _NOPS - s0loc - _HSPAN
    pltpu.sync_copy(ford_hbm.at[pl.ds(joff, _HSPAN)], f_v)
    pltpu.sync_copy(tord_hbm.at[pl.ds(joff, _HSPAN)], t_v)

    def group_body(gi, _):
        jg = _HGRP - 1 - gi
        f16 = jnp.flip(f_v[pl.ds(jg * 16, 16)])
        t16 = jnp.flip(t_v[pl.ds(jg * 16, 16)])
        q = lane
        found = lane < 0
        for d in range(1, 16):
            tm = _take16(t16, jnp.maximum(lane - d, 0))
            match = (tm == f16) & (lane >= d) & (~found)
            q = jnp.where(match, lane - d, q)
            found = found | match
        for _r in range(4):
            q = _take16(q, q)
        killed = lane < 0
        for d in range(1, 16):
            tl = _take16(t16, jnp.minimum(lane + d, 15))
            killed = killed | ((tl == t16) & (lane < 16 - d))
        fr_v[pl.ds(gi * 16, 16)] = f16
        tr_v[pl.ds(gi * 16, 16)] = t16
        qp_v[pl.ds(gi * 16, 16)] = q + (killed.astype(jnp.int32) << 8)
        return 0

    lax.fori_loop(0, _HGRP, group_body, 0)
    soff = wid * _HSPAN
    pltpu.sync_copy(fr_v, fr_hbm.at[pl.ds(soff, _HSPAN)])
    pltpu.sync_copy(tr_v, tr_hbm.at[pl.ds(soff, _HSPAN)])
    pltpu.sync_copy(qp_v, qp_hbm.at[pl.ds(soff, _HSPAN)])


@functools.partial(
    pl.kernel,
    out_type=jax.ShapeDtypeStruct((_B * _NB,), jnp.int32),
    mesh=_mesh,
    scratch_types=[
        pltpu.VMEM((_NB,), jnp.int32),
        pltpu.VMEM((_CHUNK,), jnp.int32),
        pltpu.VMEM((_CHUNK,), jnp.int32),
        pltpu.VMEM((_CHUNK,), jnp.int32),
        pltpu.VMEM((_MCHUNK,), jnp.int32),
    ],
    compiler_params=_params,
)
def _chain_kernel(midx_hbm, fr_hbm, tr_hbm, qp_hbm, g_hbm,
                  src_v, f_v, t_v, qp_v, m_v):
    wid = lax.axis_index("s") * 2 + lax.axis_index("c")
    lane = lax.iota(jnp.int32, 16)

    @pl.when(wid < _B)
    def _():
        b = wid
        sent = b * _EXT + _NS

        def init_body(i, _):
            for u in range(_UNROLL):
                src_v[pl.ds((i * _UNROLL + u) * 16, 16)] = (
                    jnp.full((16,), 0, jnp.int32) + sent
                )
            return 0

        lax.fori_loop(0, _NB // 16 // _UNROLL, init_body, 0)

        for mc in range(_NS // _MCHUNK):
            pltpu.sync_copy(midx_hbm.at[pl.ds(b * _NS + mc * _MCHUNK, _MCHUNK)], m_v)
            pbase = b * _EXT + mc * _MCHUNK

            def mask_body(i, _):
                for u in range(_UNROLL):
                    k = i * _UNROLL + u
                    m16 = m_v[pl.ds(k * 16, 16)]
                    plsc.store_scatter(src_v, [m16], pbase + k * 16 + lane)
                return 0

            lax.fori_loop(0, _MCHUNK // 16 // _UNROLL, mask_body, 0)

        for ci in range(_NOPS // _CHUNK):
            off = b * _NOPS + ci * _CHUNK
            pltpu.sync_copy(fr_hbm.at[pl.ds(off, _CHUNK)], f_v)
            pltpu.sync_copy(tr_hbm.at[pl.ds(off, _CHUNK)], t_v)
            pltpu.sync_copy(qp_hbm.at[pl.ds(off, _CHUNK)], qp_v)

            def group_body(gi, _):
                for u in range(_UNROLL):
                    k = gi * _UNROLL + u
                    f16 = f_v[pl.ds(k * 16, 16)]
                    t16 = t_v[pl.ds(k * 16, 16)]
                    qp16 = qp_v[pl.ds(k * 16, 16)]
                    srcvals = plsc.load_gather(src_v, [f16])
                    vals = _take16(srcvals, qp16 & 15)
                    plsc.store_scatter(src_v, [t16], vals, mask=qp16 < 256)
                return 0

            lax.fori_loop(0, _NGRP // _UNROLL, group_body, 0)

        pltpu.sync_copy(src_v, g_hbm.at[pl.ds(b * _NB, _NB)])


@functools.partial(
    pl.kernel,
    out_type=jax.ShapeDtypeStruct((_B * _NB, _D), jnp.float32),
    mesh=_mesh,
    scratch_types=[
        pltpu.VMEM((_NSLOT, _GCHUNK), jnp.int32),
        pltpu.VMEM((_NSLOT, _GCHUNK, _D), jnp.float32),
    ]
    + [pltpu.SemaphoreType.DMA] * (2 * _NSLOT),
)
def _gather_kernel(imgext_hbm, g_hbm, out_hbm, idx_v, rows_v, *sems):
    gsems = sems[:_NSLOT]
    osems = sems[_NSLOT:]
    wid = lax.axis_index("s") * 2 + lax.axis_index("c")
    base = wid * _ROWS_PER_TILE
    nchunks = _ROWS_PER_TILE // _GCHUNK

    def start_gather(c, s):
        pltpu.sync_copy(g_hbm.at[pl.ds(base + c * _GCHUNK, _GCHUNK)], idx_v.at[s])
        pltpu.async_copy(imgext_hbm.at[idx_v.at[s]], rows_v.at[s], gsems[s])

    for p in range(2):
        start_gather(p, p)

    def chunk_body(i, _):
        for p in range(_NSLOT):
            c = i * _NSLOT + p
            s = p
            sn = (p + 2) % _NSLOT
            pltpu.make_async_copy(
                imgext_hbm.at[idx_v.at[s]], rows_v.at[s], gsems[s]
            ).wait()
            pltpu.async_copy(
                rows_v.at[s], out_hbm.at[pl.ds(base + c * _GCHUNK, _GCHUNK)],
                osems[s],
            )
            cn = c + 2

            @pl.when(cn < nchunks)
            def _():
                @pl.when(c >= 2)
                def _():
                    pltpu.make_async_copy(
                        rows_v.at[sn],
                        out_hbm.at[pl.ds(base, _GCHUNK)],
                        osems[sn],
                    ).wait()

                start_gather(cn, sn)

        return 0

    lax.fori_loop(0, nchunks // _NSLOT, chunk_body, 0)
    for p in range(_NSLOT):
        s = (nchunks - 1 - p) % _NSLOT
        pltpu.make_async_copy(
            rows_v.at[s], out_hbm.at[pl.ds(base, _GCHUNK)], osems[s]
        ).wait()


def kernel(images, mask_idx, order):
    imgext = jnp.concatenate(
        [images, jnp.zeros((_B, 1, _D), images.dtype)], axis=1
    ).reshape(_B * _EXT, _D)
    midx = mask_idx.astype(jnp.int32).reshape(-1)
    ford = order[:, 0, :].astype(jnp.int32).reshape(-1)
    tord = order[:, 1, :].astype(jnp.int32).reshape(-1)
    fr, tr, qp = _hazard_kernel(ford, tord)
    g = _chain_kernel(midx, fr, tr, qp)
    out = _gather_kernel(imgext, g)
    return out.reshape(_B, _NB, _D)
